# Initial kernel scaffold; baseline (speedup 1.0000x reference)
import jax
import jax.numpy as jnp
from jax.experimental import pallas as pl


def kernel(x, e_i_ts):
    raise NotImplementedError("write your pallas kernel here")


# speedup vs baseline: 1.0500x; 1.0500x over previous
import functools

import jax
import jax.numpy as jnp
from jax.experimental import pallas as pl
from jax.experimental.pallas import tpu as pltpu

EMB_D = 64
NUM_K = 1024
ROWS = 1024


def _vq_block(x_ref, e_ref, et_ref, q_ref, idx_ref, loss_ref):
    i = pl.program_id(0)
    nblk = pl.num_programs(0)

    xb = x_ref[...]
    e = e_ref[...]

    xsq = jnp.sum(xb * xb, axis=1, keepdims=True)
    esq = jnp.sum(e * e, axis=0, keepdims=True)
    ip = jnp.dot(xb, e, preferred_element_type=jnp.float32)
    d = xsq - 2.0 * ip + esq

    dmin = jnp.min(d, axis=1, keepdims=True)
    kiota = jax.lax.broadcasted_iota(jnp.int32, (ROWS, NUM_K), 1)
    hit = d == dmin
    idx = jnp.min(jnp.where(hit, kiota, NUM_K), axis=1)

    onehot = (kiota == idx[:, None]).astype(jnp.float32)
    q = jax.lax.dot_general(
        onehot, et_ref[...],
        dimension_numbers=(((1,), (0,)), ((), ())),
        precision=jax.lax.Precision.HIGHEST,
        preferred_element_type=jnp.float32,
    )

    q_ref[...] = xb + (q - xb)
    idx_ref[...] = idx.reshape(1, 1, ROWS)

    diff = xb - q
    partial = jnp.sum(diff * diff)

    @pl.when(i == 0)
    def _():
        loss_ref[0, 0] = 0.0

    loss_ref[0, 0] += partial


def kernel(x, e_i_ts):
    B, C, H, W = x.shape
    n = B * H * W
    nblk = n // ROWS

    flat_x = jnp.transpose(x, (0, 2, 3, 1)).reshape(n, C)
    et = e_i_ts.T

    q_flat, idx3, loss_acc = pl.pallas_call(
        _vq_block,
        grid=(nblk,),
        in_specs=[
            pl.BlockSpec((ROWS, C), lambda i: (i, 0)),
            pl.BlockSpec((C, NUM_K), lambda i: (0, 0)),
            pl.BlockSpec((NUM_K, C), lambda i: (0, 0)),
        ],
        out_specs=[
            pl.BlockSpec((ROWS, C), lambda i: (i, 0)),
            pl.BlockSpec((1, 1, ROWS), lambda i: (i, 0, 0)),
            pl.BlockSpec((1, 1), lambda i: (0, 0), memory_space=pltpu.SMEM),
        ],
        out_shape=[
            jax.ShapeDtypeStruct((n, C), jnp.float32),
            jax.ShapeDtypeStruct((nblk, 1, ROWS), jnp.int32),
            jax.ShapeDtypeStruct((1, 1), jnp.float32),
        ],
    )(flat_x, e_i_ts, et)

    quantized_x_st = jnp.transpose(q_flat.reshape(B, H, W, C), (0, 3, 1, 2))
    loss = loss_acc[0, 0] / jnp.float32(n * C)
    encoding_indices = idx3.reshape(B, H * W)
    return (quantized_x_st, loss, loss, encoding_indices)


# speedup vs baseline: 1.6213x; 1.5441x over previous
import jax
import jax.numpy as jnp
from jax.experimental import pallas as pl
from jax.experimental.pallas import tpu as pltpu

EMB_D = 64
NUM_K = 1024
ROWS = 1024


def _vq_block(x_ref, e_ref, etb_ref, q_ref, idx_ref, loss_ref):
    i = pl.program_id(0)

    xc = x_ref[0]
    xt = xc.T

    e = e_ref[...]
    e2 = e * (-2.0)
    esq = jnp.sum(e * e, axis=0, keepdims=True)

    xsq = jnp.sum(xt * xt, axis=1, keepdims=True)
    ip2 = jnp.dot(xt, e2, preferred_element_type=jnp.float32)
    d = xsq + ip2 + esq

    idx = jnp.argmin(d, axis=1).astype(jnp.int32)

    kiota = jax.lax.broadcasted_iota(jnp.int32, (ROWS, NUM_K), 1)
    onehot = (kiota == idx[:, None]).astype(jnp.bfloat16)
    q = jnp.dot(onehot, etb_ref[...], preferred_element_type=jnp.float32)

    q_ref[0] = (xt + (q - xt)).T
    idx_ref[...] = idx.reshape(1, 1, ROWS)

    diff = xt - q
    partial = jnp.sum(diff * diff)

    @pl.when(i == 0)
    def _():
        loss_ref[0, 0] = 0.0

    loss_ref[0, 0] += partial


def kernel(x, e_i_ts):
    B, C, H, W = x.shape
    n = B * H * W

    xr = x.reshape(B, C, H * W)
    etb = e_i_ts.T.astype(jnp.bfloat16)

    q_r, idx3, loss_acc = pl.pallas_call(
        _vq_block,
        grid=(B,),
        in_specs=[
            pl.BlockSpec((1, C, ROWS), lambda i: (i, 0, 0)),
            pl.BlockSpec((C, NUM_K), lambda i: (0, 0)),
            pl.BlockSpec((NUM_K, C), lambda i: (0, 0)),
        ],
        out_specs=[
            pl.BlockSpec((1, C, ROWS), lambda i: (i, 0, 0)),
            pl.BlockSpec((1, 1, ROWS), lambda i: (i, 0, 0)),
            pl.BlockSpec((1, 1), lambda i: (0, 0), memory_space=pltpu.SMEM),
        ],
        out_shape=[
            jax.ShapeDtypeStruct((B, C, H * W), jnp.float32),
            jax.ShapeDtypeStruct((B, 1, ROWS), jnp.int32),
            jax.ShapeDtypeStruct((1, 1), jnp.float32),
        ],
    )(xr, e_i_ts, etb)

    quantized_x_st = q_r.reshape(B, C, H, W)
    loss = loss_acc[0, 0] / jnp.float32(n * C)
    encoding_indices = idx3.reshape(B, H * W)
    return (quantized_x_st, loss, loss, encoding_indices)


# speedup vs baseline: 1.8364x; 1.1327x over previous
import jax
import jax.numpy as jnp
from jax.experimental import pallas as pl
from jax.experimental.pallas import tpu as pltpu

EMB_D = 64
NUM_K = 1024
BPB = 2
HW = 1024
ROWS = BPB * HW


def _vq_block(x_ref, e_ref, etb_ref, q_ref, idx_ref, loss_ref):
    i = pl.program_id(0)

    xc = x_ref[...]
    xt = jnp.transpose(xc, (0, 2, 1)).reshape(ROWS, EMB_D)

    e = e_ref[...]
    e2 = e * (-2.0)
    esq = jnp.sum(e * e, axis=0, keepdims=True)

    xsq = jnp.sum(xt * xt, axis=1, keepdims=True)
    ip2 = jnp.dot(xt, e2, preferred_element_type=jnp.float32)
    d = xsq + ip2 + esq

    idx = jnp.argmin(d, axis=1).astype(jnp.int32)

    kiota = jax.lax.broadcasted_iota(jnp.int32, (ROWS, NUM_K), 1)
    onehot = (kiota == idx[:, None]).astype(jnp.bfloat16)
    q = jnp.dot(onehot, etb_ref[...], preferred_element_type=jnp.float32)

    qst = (xt + (q - xt)).reshape(BPB, HW, EMB_D)
    q_ref[...] = jnp.transpose(qst, (0, 2, 1))
    idx_ref[...] = idx.reshape(BPB, 1, HW)

    diff = xt - q
    partial = jnp.sum(diff * diff)

    @pl.when(i == 0)
    def _():
        loss_ref[0, 0] = 0.0

    loss_ref[0, 0] += partial


def kernel(x, e_i_ts):
    B, C, H, W = x.shape
    n = B * H * W

    xr = x.reshape(B, C, H * W)
    etb = e_i_ts.T.astype(jnp.bfloat16)

    q_r, idx3, loss_acc = pl.pallas_call(
        _vq_block,
        grid=(B // BPB,),
        in_specs=[
            pl.BlockSpec((BPB, C, HW), lambda i: (i, 0, 0)),
            pl.BlockSpec((C, NUM_K), lambda i: (0, 0)),
            pl.BlockSpec((NUM_K, C), lambda i: (0, 0)),
        ],
        out_specs=[
            pl.BlockSpec((BPB, C, HW), lambda i: (i, 0, 0)),
            pl.BlockSpec((BPB, 1, HW), lambda i: (i, 0, 0)),
            pl.BlockSpec((1, 1), lambda i: (0, 0), memory_space=pltpu.SMEM),
        ],
        out_shape=[
            jax.ShapeDtypeStruct((B, C, H * W), jnp.float32),
            jax.ShapeDtypeStruct((B, 1, HW), jnp.int32),
            jax.ShapeDtypeStruct((1, 1), jnp.float32),
        ],
    )(xr, e_i_ts, etb)

    quantized_x_st = q_r.reshape(B, C, H, W)
    loss = loss_acc[0, 0] / jnp.float32(n * C)
    encoding_indices = idx3.reshape(B, H * W)
    return (quantized_x_st, loss, loss, encoding_indices)


# speedup vs baseline: 1.9039x; 1.0368x over previous
import jax
import jax.numpy as jnp
from jax.experimental import pallas as pl
from jax.experimental.pallas import tpu as pltpu

EMB_D = 64
NUM_K = 1024
BPB = 4
HW = 1024
ROWS = BPB * HW


def _vq_block(x_ref, e_ref, etb_ref, q_ref, idx_ref, loss_ref):
    i = pl.program_id(0)

    xc = x_ref[...]
    xt = jnp.transpose(xc, (0, 2, 1)).reshape(ROWS, EMB_D)

    e = e_ref[...]
    e2 = e * (-2.0)
    esq = jnp.sum(e * e, axis=0, keepdims=True)

    xsq = jnp.sum(xt * xt, axis=1, keepdims=True)
    ip2 = jnp.dot(xt, e2, preferred_element_type=jnp.float32)
    d = xsq + ip2 + esq

    idx = jnp.argmin(d, axis=1).astype(jnp.int32)

    kiota = jax.lax.broadcasted_iota(jnp.int32, (ROWS, NUM_K), 1)
    onehot = (kiota == idx[:, None]).astype(jnp.bfloat16)
    q = jnp.dot(onehot, etb_ref[...], preferred_element_type=jnp.float32)

    qst = (xt + (q - xt)).reshape(BPB, HW, EMB_D)
    q_ref[...] = jnp.transpose(qst, (0, 2, 1))
    idx_ref[...] = idx.reshape(BPB, 1, HW)

    diff = xt - q
    partial = jnp.sum(diff * diff)

    @pl.when(i == 0)
    def _():
        loss_ref[0, 0] = 0.0

    loss_ref[0, 0] += partial


def kernel(x, e_i_ts):
    B, C, H, W = x.shape
    n = B * H * W

    xr = x.reshape(B, C, H * W)
    etb = e_i_ts.T.astype(jnp.bfloat16)

    q_r, idx3, loss_acc = pl.pallas_call(
        _vq_block,
        grid=(B // BPB,),
        in_specs=[
            pl.BlockSpec((BPB, C, HW), lambda i: (i, 0, 0)),
            pl.BlockSpec((C, NUM_K), lambda i: (0, 0)),
            pl.BlockSpec((NUM_K, C), lambda i: (0, 0)),
        ],
        out_specs=[
            pl.BlockSpec((BPB, C, HW), lambda i: (i, 0, 0)),
            pl.BlockSpec((BPB, 1, HW), lambda i: (i, 0, 0)),
            pl.BlockSpec((1, 1), lambda i: (0, 0), memory_space=pltpu.SMEM),
        ],
        out_shape=[
            jax.ShapeDtypeStruct((B, C, H * W), jnp.float32),
            jax.ShapeDtypeStruct((B, 1, HW), jnp.int32),
            jax.ShapeDtypeStruct((1, 1), jnp.float32),
        ],
    )(xr, e_i_ts, etb)

    quantized_x_st = q_r.reshape(B, C, H, W)
    loss = loss_acc[0, 0] / jnp.float32(n * C)
    encoding_indices = idx3.reshape(B, H * W)
    return (quantized_x_st, loss, loss, encoding_indices)
